# baseline (speedup 1.0000x reference)
import jax
import jax.numpy as jnp
from jax import lax
from jax.experimental import pallas as pl
from jax.experimental.pallas import tpu as pltpu
from jax.experimental.pallas import tpu_sc as plsc

B, S, D = 4, 8192, 768
ROWS = B * S
SC_ROWS = ROWS // 2
NW = 32
RPW = SC_ROWS // NW
CH = 16
NCH = RPW // CH
NL = 16
DV = D // NL
RB = 512
NBLK = ROWS // RB
TIME_ROWS = 288


def _pe_body(x_hbm, tb_hbm, pos_hbm, time_hbm, out_hbm,
             xb, tbuf, pb, ob, idxall, semL0, semL1, semS0, semS1):
    wid = lax.axis_index("s") * 2 + lax.axis_index("c")
    base = wid * RPW
    sbase = base % S
    semL = (semL0, semL1)
    semS = (semS0, semS1)

    pltpu.sync_copy(tb_hbm.at[pl.ds(base, RPW)], idxall)

    def load_descs(i, b):
        r0 = base + i * CH
        p0 = sbase + i * CH
        return (
            pltpu.make_async_copy(x_hbm.at[pl.ds(r0, CH)], xb.at[b], semL[b]),
            pltpu.make_async_copy(pos_hbm.at[pl.ds(p0, CH)], pb.at[b], semL[b]),
            pltpu.make_async_copy(
                time_hbm.at[idxall.at[pl.ds(i * CH, CH)]], tbuf.at[b], semL[b]),
        )

    def store_desc(i, b):
        r0 = base + i * CH
        return pltpu.make_async_copy(ob.at[b], out_hbm.at[pl.ds(r0, CH)],
                                     semS[b])

    def issue_loads(i, b):
        for d in load_descs(i, b):
            d.start()

    for b in (0, 1):
        issue_loads(b, b)

    def chunk(i, b):
        for d in load_descs(i, b):
            d.wait()

        @pl.when(i >= 2)
        def _():
            store_desc(i - 2, b).wait()

        x_, t_, p_, o_ = xb.at[b], tbuf.at[b], pb.at[b], ob.at[b]

        def row(c, carry):
            for j in range(DV):
                sl = pl.ds(j * NL, NL)
                o_[c, sl] = x_[c, sl] + t_[c, sl] + p_[c, sl]
            return carry

        lax.fori_loop(0, CH, row, None)
        store_desc(i, b).start()

        @pl.when(i + 2 < NCH)
        def _():
            issue_loads(i + 2, b)

    def outer(g, carry):
        chunk(2 * g, 0)
        chunk(2 * g + 1, 1)
        return carry

    lax.fori_loop(0, NCH // 2, outer, None)
    store_desc(NCH - 2, 0).wait()
    store_desc(NCH - 1, 1).wait()


@jax.jit
def _pe(x2d, tb1d, pos_table, time_table):
    mesh = plsc.VectorSubcoreMesh(core_axis_name="c", subcore_axis_name="s")
    return pl.kernel(
        _pe_body,
        mesh=mesh,
        out_type=jax.ShapeDtypeStruct((ROWS, D), jnp.float32),
        scratch_types=[
            pltpu.VMEM((2, CH, D), jnp.float32),
            pltpu.VMEM((2, CH, D), jnp.float32),
            pltpu.VMEM((2, CH, D), jnp.float32),
            pltpu.VMEM((2, CH, D), jnp.float32),
            pltpu.VMEM((RPW,), jnp.int32),
            pltpu.SemaphoreType.DMA,
            pltpu.SemaphoreType.DMA,
            pltpu.SemaphoreType.DMA,
            pltpu.SemaphoreType.DMA,
        ],
    )(x2d, tb1d, pos_table, time_table)


def _tc_body(alias_ref, x_ref, tb_ref, pos_ref, tt_ref, out_ref):
    del alias_ref
    tb_vec = tb_ref[0, 0, :]
    onehot = (tb_vec[:, None]
              == lax.broadcasted_iota(jnp.int32, (RB, TIME_ROWS), 1))
    temb = jnp.dot(onehot.astype(jnp.bfloat16), tt_ref[...],
                   preferred_element_type=jnp.float32)
    out_ref[0] = x_ref[0] + pos_ref[0] + temb


def _tc_fill(out_sc, xv, tbv, posv, tt16):
    nb = NBLK // 2
    return pl.pallas_call(
        _tc_body,
        grid=(nb,),
        in_specs=[
            pl.BlockSpec(memory_space=pltpu.MemorySpace.HBM),
            pl.BlockSpec((1, RB, D), lambda j: (nb + j, 0, 0)),
            pl.BlockSpec((1, 1, RB), lambda j: (nb + j, 0, 0)),
            pl.BlockSpec((1, RB, D), lambda j: (lax.rem(nb + j, S // RB), 0, 0)),
            pl.BlockSpec((TIME_ROWS, D), lambda j: (0, 0)),
        ],
        out_specs=pl.BlockSpec((1, RB, D), lambda j: (nb + j, 0, 0)),
        out_shape=jax.ShapeDtypeStruct((NBLK, RB, D), jnp.float32),
        input_output_aliases={0: 0},
    )(out_sc, xv, tbv, posv, tt16)


@jax.jit
def _pe_hybrid(x2d, tb1d, pos_table, time_table):
    out_sc = _pe(x2d, tb1d, pos_table, time_table)
    xv = x2d.reshape(NBLK, RB, D)
    tbv = tb1d.reshape(NBLK, 1, RB)
    posv = pos_table.reshape(S // RB, RB, D)
    tt16 = time_table.astype(jnp.bfloat16)
    return _tc_fill(out_sc.reshape(NBLK, RB, D), xv, tbv, posv, tt16)


def kernel(x, time_buckets, pos_table, time_table):
    x2d = x.reshape(ROWS, D)
    tb1d = time_buckets.astype(jnp.int32).reshape(ROWS)
    out = _pe_hybrid(x2d, tb1d, pos_table, time_table)
    return out.reshape(B, S, D)


# speedup vs baseline: 1.0836x; 1.0836x over previous
import jax
import jax.numpy as jnp
from jax import lax
from jax.experimental import pallas as pl
from jax.experimental.pallas import tpu as pltpu
from jax.experimental.pallas import tpu_sc as plsc

B, S, D = 4, 8192, 768
ROWS = B * S
SC_ROWS = ROWS // 2
NW = 32
RPW = SC_ROWS // NW
CH = 16
NCH = RPW // CH
NL = 16
DV = D // NL
RB = 512
NBLK = ROWS // RB
TIME_ROWS = 288


def _pe_body(x_hbm, tb_hbm, pos_hbm, time_hbm, out_hbm,
             xb, tbuf, pb, ob, idxall, semL0, semL1, semS0, semS1):
    wid = lax.axis_index("s") * 2 + lax.axis_index("c")
    base = wid * RPW
    sbase = base % S
    semL = (semL0, semL1)
    semS = (semS0, semS1)

    pltpu.sync_copy(tb_hbm.at[pl.ds(base, RPW)], idxall)

    def load_descs(i, b):
        r0 = base + i * CH
        p0 = sbase + i * CH
        return (
            pltpu.make_async_copy(x_hbm.at[pl.ds(r0, CH)], xb.at[b], semL[b]),
            pltpu.make_async_copy(pos_hbm.at[pl.ds(p0, CH)], pb.at[b], semL[b]),
            pltpu.make_async_copy(
                time_hbm.at[idxall.at[pl.ds(i * CH, CH)]], tbuf.at[b], semL[b]),
        )

    def store_desc(i, b):
        r0 = base + i * CH
        return pltpu.make_async_copy(ob.at[b], out_hbm.at[pl.ds(r0, CH)],
                                     semS[b])

    def issue_loads(i, b):
        for d in load_descs(i, b):
            d.start()

    for b in (0, 1):
        issue_loads(b, b)

    def chunk(i, b):
        for d in load_descs(i, b):
            d.wait()

        @pl.when(i >= 2)
        def _():
            store_desc(i - 2, b).wait()

        x_, t_, p_, o_ = xb.at[b], tbuf.at[b], pb.at[b], ob.at[b]

        def row(c, carry):
            for j in range(DV):
                sl = pl.ds(j * NL, NL)
                o_[c, sl] = x_[c, sl] + t_[c, sl] + p_[c, sl]
            return carry

        lax.fori_loop(0, CH, row, None)
        store_desc(i, b).start()

        @pl.when(i + 2 < NCH)
        def _():
            issue_loads(i + 2, b)

    def outer(g, carry):
        chunk(2 * g, 0)
        chunk(2 * g + 1, 1)
        return carry

    lax.fori_loop(0, NCH // 2, outer, None)
    store_desc(NCH - 2, 0).wait()
    store_desc(NCH - 1, 1).wait()


@jax.jit
def _pe(x2d, tb1d, pos_table, time_table):
    mesh = plsc.VectorSubcoreMesh(core_axis_name="c", subcore_axis_name="s")
    return pl.kernel(
        _pe_body,
        mesh=mesh,
        out_type=jax.ShapeDtypeStruct((ROWS, D), jnp.float32),
        scratch_types=[
            pltpu.VMEM((2, CH, D), jnp.float32),
            pltpu.VMEM((2, CH, D), jnp.float32),
            pltpu.VMEM((2, CH, D), jnp.float32),
            pltpu.VMEM((2, CH, D), jnp.float32),
            pltpu.VMEM((RPW,), jnp.int32),
            pltpu.SemaphoreType.DMA,
            pltpu.SemaphoreType.DMA,
            pltpu.SemaphoreType.DMA,
            pltpu.SemaphoreType.DMA,
        ],
    )(x2d, tb1d, pos_table, time_table)


def _tc_body(alias_ref, x_ref, tb_ref, pos_ref, tt_ref, out_ref):
    del alias_ref
    for k in range(2):
        tb_vec = tb_ref[k, 0, 0, :]
        onehot = (tb_vec[:, None]
                  == lax.broadcasted_iota(jnp.int32, (RB, TIME_ROWS), 1))
        temb = jnp.dot(onehot.astype(jnp.bfloat16), tt_ref[...],
                       preferred_element_type=jnp.float32)
        out_ref[k, 0] = x_ref[k, 0] + pos_ref[0] + temb


def _tc_fill(out_sc, xv, tbv, posv, tt16):
    nsb = S // RB
    return pl.pallas_call(
        _tc_body,
        grid=(nsb,),
        in_specs=[
            pl.BlockSpec(memory_space=pltpu.MemorySpace.HBM),
            pl.BlockSpec((2, 1, RB, D), lambda j: (1, j, 0, 0)),
            pl.BlockSpec((2, 1, 1, RB), lambda j: (1, j, 0, 0)),
            pl.BlockSpec((1, RB, D), lambda j: (j, 0, 0)),
            pl.BlockSpec((TIME_ROWS, D), lambda j: (0, 0)),
        ],
        out_specs=pl.BlockSpec((2, 1, RB, D), lambda j: (1, j, 0, 0)),
        out_shape=jax.ShapeDtypeStruct((B, S // RB, RB, D), jnp.float32),
        input_output_aliases={0: 0},
    )(out_sc, xv, tbv, posv, tt16)


@jax.jit
def _pe_hybrid(x2d, tb1d, pos_table, time_table):
    out_sc = _pe(x2d, tb1d, pos_table, time_table)
    xv = x2d.reshape(B, S // RB, RB, D)
    tbv = tb1d.reshape(B, S // RB, 1, RB)
    posv = pos_table.reshape(S // RB, RB, D)
    tt16 = time_table.astype(jnp.bfloat16)
    return _tc_fill(out_sc.reshape(B, S // RB, RB, D), xv, tbv, posv, tt16)


def kernel(x, time_buckets, pos_table, time_table):
    x2d = x.reshape(ROWS, D)
    tb1d = time_buckets.astype(jnp.int32).reshape(ROWS)
    out = _pe_hybrid(x2d, tb1d, pos_table, time_table)
    return out.reshape(B, S, D)
